# baseline (speedup 1.0000x reference)
import functools

import numpy as np
import jax
import jax.numpy as jnp
from jax import lax
from jax.experimental import pallas as pl
from jax.experimental.pallas import tpu as pltpu
from jax.experimental.pallas import tpu_sc as plsc

D_MODEL = 1024
LANES = 16
NUM_CORES = 2
NUM_SUBCORES = 16
NUM_WORKERS = NUM_CORES * NUM_SUBCORES


@functools.lru_cache(maxsize=None)
def _pos_encoding(seq_len: int) -> np.ndarray:
    pos = np.arange(seq_len, dtype=np.float32)[:, None]
    den = np.exp(
        (-np.arange(0, D_MODEL, 2, dtype=np.float32)) *
        np.float32(np.log(10000.0) / D_MODEL))
    pe = np.zeros((seq_len, D_MODEL), dtype=np.float32)
    pe[:, 0::2] = np.sin(pos * den)
    pe[:, 1::2] = np.cos(pos * den)
    return pe


@functools.lru_cache(maxsize=None)
def _build_sc_call(seq: int, batch: int):
    num_rows = seq * batch
    rows_per_w = num_rows // NUM_WORKERS
    cs = 4
    cr = cs * batch
    nchunk = rows_per_w // cr
    s_per_w = rows_per_w // batch
    ndb = D_MODEL // LANES
    scale = float(np.sqrt(D_MODEL))
    nbuf = 4
    lookahead = 3

    mesh = plsc.VectorSubcoreMesh(
        core_axis_name="c", subcore_axis_name="s")

    scratch = (
        [pltpu.VMEM((rows_per_w,), jnp.int32)]
        + [pltpu.VMEM((cr, D_MODEL), jnp.float32) for _ in range(nbuf)]
        + [pltpu.VMEM((cs, D_MODEL), jnp.float32) for _ in range(nbuf)]
        + [pltpu.SemaphoreType.DMA for _ in range(3 * nbuf)]
    )

    @functools.partial(
        pl.kernel,
        mesh=mesh,
        out_type=jax.ShapeDtypeStruct((seq, batch, D_MODEL), jnp.float32),
        scratch_types=scratch,
    )
    def embed(x_hbm, pe_hbm, table_hbm, out_hbm, idx_v, *bufs):
        rows_b = bufs[:nbuf]
        pe_b = bufs[nbuf:2 * nbuf]
        gsem = bufs[2 * nbuf:3 * nbuf]
        psem = bufs[3 * nbuf:4 * nbuf]
        ssem = bufs[4 * nbuf:5 * nbuf]

        wid = lax.axis_index("s") * NUM_CORES + lax.axis_index("c")
        base = wid * rows_per_w
        pltpu.sync_copy(x_hbm.at[pl.ds(base, rows_per_w)], idx_v)

        def gather_descs(c, k):
            return (
                pltpu.make_async_copy(
                    table_hbm.at[idx_v.at[pl.ds(c * cr, cr)]],
                    rows_b[k], gsem[k]),
                pltpu.make_async_copy(
                    pe_hbm.at[pl.ds(wid * s_per_w + c * cs, cs)],
                    pe_b[k], psem[k]),
            )

        def store_descs(c, k):
            s0 = wid * s_per_w + c * cs
            return [
                pltpu.make_async_copy(
                    rows_b[k].at[pl.ds(si * batch, batch)],
                    out_hbm.at[s0 + si], ssem[k])
                for si in range(cs)
            ]

        def start_fetch(c, k):
            g, p = gather_descs(c, k)
            g.start()
            p.start()

        for k in range(lookahead):
            start_fetch(k, k)

        def step(c, k):
            cp = c + lookahead
            kp = (k + lookahead) % nbuf

            @pl.when(cp < nchunk)
            def _prefetch():
                @pl.when(cp >= nbuf)
                def _drain_store():
                    for d in store_descs(cp - nbuf, kp):
                        d.wait()
                start_fetch(cp, kp)

            g, p = gather_descs(c, k)
            g.wait()
            p.wait()

            rows_v, pe_v = rows_b[k], pe_b[k]

            def row_body(si, carry2):
                def d_body(db2, carry3):
                    for u in range(2):
                        dd = (db2 * 2 + u) * LANES
                        pe_vec = pe_v[si, pl.ds(dd, LANES)]
                        for b in range(batch):
                            r = si * batch + b
                            rows_v[r, pl.ds(dd, LANES)] = (
                                rows_v[r, pl.ds(dd, LANES)] * scale + pe_vec)
                    return carry3
                return lax.fori_loop(0, ndb // 2, d_body, carry2)
            lax.fori_loop(0, cs, row_body, 0)

            for d in store_descs(c, k):
                d.start()

        def group_body(grp, carry):
            c0 = grp * nbuf
            for k in range(nbuf):
                step(c0 + k, k)
            return carry
        lax.fori_loop(0, nchunk // nbuf, group_body, 0)

        for k in range(nbuf):
            for d in store_descs(nchunk - nbuf + k, k):
                d.wait()

    return embed


def kernel(x, table):
    seq, batch = x.shape
    xf = x.reshape(seq * batch).astype(jnp.int32)
    pe = jnp.asarray(_pos_encoding(seq))
    return _build_sc_call(seq, batch)(xf, pe, table)


# speedup vs baseline: 1.0872x; 1.0872x over previous
import functools

import numpy as np
import jax
import jax.numpy as jnp
from jax import lax
from jax.experimental import pallas as pl
from jax.experimental.pallas import tpu as pltpu
from jax.experimental.pallas import tpu_sc as plsc

D_MODEL = 1024
LANES = 16
NUM_CORES = 2
NUM_SUBCORES = 16
NUM_WORKERS = NUM_CORES * NUM_SUBCORES


@functools.lru_cache(maxsize=None)
def _pos_encoding(seq_len: int) -> np.ndarray:
    pos = np.arange(seq_len, dtype=np.float32)[:, None]
    den = np.exp(
        (-np.arange(0, D_MODEL, 2, dtype=np.float32)) *
        np.float32(np.log(10000.0) / D_MODEL))
    pe = np.zeros((seq_len, D_MODEL), dtype=np.float32)
    pe[:, 0::2] = np.sin(pos * den)
    pe[:, 1::2] = np.cos(pos * den)
    return pe


@functools.lru_cache(maxsize=None)
def _build_sc_call(seq: int, batch: int):
    num_rows = seq * batch
    rows_per_w = num_rows // NUM_WORKERS
    cs = 4
    cr = cs * batch
    nchunk = rows_per_w // cr
    s_per_w = rows_per_w // batch
    ndb = D_MODEL // LANES
    scale = float(np.sqrt(D_MODEL))
    nbuf = 5
    lookahead = 3
    groups = nchunk // nbuf
    rem = nchunk % nbuf

    mesh = plsc.VectorSubcoreMesh(
        core_axis_name="c", subcore_axis_name="s")

    scratch = (
        [pltpu.VMEM((rows_per_w,), jnp.int32)]
        + [pltpu.VMEM((cr, D_MODEL), jnp.float32) for _ in range(nbuf)]
        + [pltpu.VMEM((cs, D_MODEL), jnp.float32) for _ in range(nbuf)]
        + [pltpu.SemaphoreType.DMA for _ in range(3 * nbuf)]
    )

    @functools.partial(
        pl.kernel,
        mesh=mesh,
        out_type=jax.ShapeDtypeStruct((seq, batch, D_MODEL), jnp.float32),
        scratch_types=scratch,
    )
    def embed(x_hbm, pe_hbm, table_hbm, out_hbm, idx_v, *bufs):
        rows_b = bufs[:nbuf]
        pe_b = bufs[nbuf:2 * nbuf]
        gsem = bufs[2 * nbuf:3 * nbuf]
        psem = bufs[3 * nbuf:4 * nbuf]
        ssem = bufs[4 * nbuf:5 * nbuf]

        wid = lax.axis_index("s") * NUM_CORES + lax.axis_index("c")
        base = wid * rows_per_w
        pltpu.sync_copy(x_hbm.at[pl.ds(base, rows_per_w)], idx_v)

        def gather_descs(c, k):
            return (
                pltpu.make_async_copy(
                    table_hbm.at[idx_v.at[pl.ds(c * cr, cr)]],
                    rows_b[k], gsem[k]),
                pltpu.make_async_copy(
                    pe_hbm.at[pl.ds(wid * s_per_w + c * cs, cs)],
                    pe_b[k], psem[k]),
            )

        def store_descs(c, k):
            s0 = wid * s_per_w + c * cs
            return [
                pltpu.make_async_copy(
                    rows_b[k].at[pl.ds(si * batch, batch)],
                    out_hbm.at[s0 + si], ssem[k])
                for si in range(cs)
            ]

        def start_fetch(c, k):
            g, p = gather_descs(c, k)
            g.start()
            p.start()

        for k in range(lookahead):
            start_fetch(k, k)

        def step(c, k):
            cp = c + lookahead
            kp = (k + lookahead) % nbuf

            @pl.when(cp < nchunk)
            def _prefetch():
                @pl.when(cp >= nbuf)
                def _drain_store():
                    for d in store_descs(cp - nbuf, kp):
                        d.wait()
                start_fetch(cp, kp)

            g, p = gather_descs(c, k)
            g.wait()
            p.wait()

            rows_v, pe_v = rows_b[k], pe_b[k]

            def row_body(si, carry2):
                def d_body(db2, carry3):
                    for u in range(2):
                        dd = (db2 * 2 + u) * LANES
                        pe_vec = pe_v[si, pl.ds(dd, LANES)]
                        for b in range(batch):
                            r = si * batch + b
                            rows_v[r, pl.ds(dd, LANES)] = (
                                rows_v[r, pl.ds(dd, LANES)] * scale + pe_vec)
                    return carry3
                return lax.fori_loop(0, ndb // 2, d_body, carry2)
            lax.fori_loop(0, cs, row_body, 0)

            for d in store_descs(c, k):
                d.start()

        def group_body(grp, carry):
            c0 = grp * nbuf
            for k in range(nbuf):
                step(c0 + k, k)
            return carry
        lax.fori_loop(0, groups, group_body, 0)

        for j in range(rem):
            step(groups * nbuf + j, j)

        for c in range(nchunk - min(nbuf, nchunk), nchunk):
            for d in store_descs(c, c % nbuf):
                d.wait()

    return embed


def kernel(x, table):
    seq, batch = x.shape
    xf = x.reshape(seq * batch).astype(jnp.int32)
    pe = jnp.asarray(_pos_encoding(seq))
    return _build_sc_call(seq, batch)(xf, pe, table)


# speedup vs baseline: 1.3236x; 1.2174x over previous
import functools

import numpy as np
import jax
import jax.numpy as jnp
from jax import lax
from jax.experimental import pallas as pl
from jax.experimental.pallas import tpu as pltpu
from jax.experimental.pallas import tpu_sc as plsc

D_MODEL = 1024
LANES = 16
NUM_CORES = 2
NUM_SUBCORES = 16
NUM_WORKERS = NUM_CORES * NUM_SUBCORES


@functools.lru_cache(maxsize=None)
def _pos_encoding(seq_len: int) -> np.ndarray:
    pos = np.arange(seq_len, dtype=np.float32)[:, None]
    den = np.exp(
        (-np.arange(0, D_MODEL, 2, dtype=np.float32)) *
        np.float32(np.log(10000.0) / D_MODEL))
    pe = np.zeros((seq_len, D_MODEL), dtype=np.float32)
    pe[:, 0::2] = np.sin(pos * den)
    pe[:, 1::2] = np.cos(pos * den)
    return pe


@functools.lru_cache(maxsize=None)
def _build_sc_call(seq: int, batch: int):
    num_rows = seq * batch
    rows_per_w = num_rows // NUM_WORKERS
    cs = 4
    cr = cs * batch
    nchunk = rows_per_w // cr
    s_per_w = rows_per_w // batch
    ndb = D_MODEL // LANES
    scale = float(np.sqrt(D_MODEL))
    nbuf = 4
    lookahead = 2
    groups = nchunk // nbuf
    rem = nchunk % nbuf

    mesh = plsc.VectorSubcoreMesh(
        core_axis_name="c", subcore_axis_name="s")

    scratch = (
        [pltpu.VMEM((rows_per_w,), jnp.int32)]
        + [pltpu.VMEM((cr, D_MODEL), jnp.float32) for _ in range(nbuf)]
        + [pltpu.VMEM((cs, D_MODEL), jnp.float32) for _ in range(nbuf)]
        + [pltpu.SemaphoreType.DMA for _ in range(3 * nbuf)]
    )

    @functools.partial(
        pl.kernel,
        mesh=mesh,
        out_type=jax.ShapeDtypeStruct((seq, batch, D_MODEL), jnp.float32),
        scratch_types=scratch,
    )
    def embed(x_hbm, pe_hbm, table_hbm, out_hbm, idx_v, *bufs):
        rows_b = bufs[:nbuf]
        pe_b = bufs[nbuf:2 * nbuf]
        gsem = bufs[2 * nbuf:3 * nbuf]
        psem = bufs[3 * nbuf:4 * nbuf]
        ssem = bufs[4 * nbuf:5 * nbuf]

        wid = lax.axis_index("s") * NUM_CORES + lax.axis_index("c")
        base = wid * rows_per_w
        pltpu.sync_copy(x_hbm.at[pl.ds(base, rows_per_w)], idx_v)

        def gather_descs(c, k):
            return (
                pltpu.make_async_copy(
                    table_hbm.at[idx_v.at[pl.ds(c * cr, cr)]],
                    rows_b[k], gsem[k]),
                pltpu.make_async_copy(
                    pe_hbm.at[pl.ds(wid * s_per_w + c * cs, cs)],
                    pe_b[k], psem[k]),
            )

        def store_descs(c, k):
            s0 = wid * s_per_w + c * cs
            return [
                pltpu.make_async_copy(
                    rows_b[k].at[pl.ds(si * batch, batch)],
                    out_hbm.at[s0 + si], ssem[k])
                for si in range(cs)
            ]

        def start_fetch(c, k):
            g, p = gather_descs(c, k)
            g.start()
            p.start()

        for k in range(lookahead):
            start_fetch(k, k)

        def step(c, k):
            cp = c + lookahead
            kp = (k + lookahead) % nbuf

            @pl.when(cp < nchunk)
            def _prefetch():
                @pl.when(cp >= nbuf)
                def _drain_store():
                    for d in store_descs(cp - nbuf, kp):
                        d.wait()
                start_fetch(cp, kp)

            g, p = gather_descs(c, k)
            g.wait()
            p.wait()

            rows_v, pe_v = rows_b[k], pe_b[k]

            for si in range(cs):
                @plsc.parallel_loop(0, ndb // 2, unroll=2)
                def d_body(db2, _si=si):
                    for u in range(2):
                        dd = (db2 * 2 + u) * LANES
                        pe_vec = pe_v[_si, pl.ds(dd, LANES)]
                        for b in range(batch):
                            r = _si * batch + b
                            rows_v[r, pl.ds(dd, LANES)] = (
                                rows_v[r, pl.ds(dd, LANES)] * scale + pe_vec)

            for d in store_descs(c, k):
                d.start()

        def group_body(grp, carry):
            c0 = grp * nbuf
            for k in range(nbuf):
                step(c0 + k, k)
            return carry
        lax.fori_loop(0, groups, group_body, 0)

        for j in range(rem):
            step(groups * nbuf + j, j)

        for c in range(nchunk - min(nbuf, nchunk), nchunk):
            for d in store_descs(c, c % nbuf):
                d.wait()

    return embed


def kernel(x, table):
    seq, batch = x.shape
    xf = x.reshape(seq * batch).astype(jnp.int32)
    pe = jnp.asarray(_pos_encoding(seq))
    return _build_sc_call(seq, batch)(xf, pe, table)


# speedup vs baseline: 1.4860x; 1.1227x over previous
import functools

import numpy as np
import jax
import jax.numpy as jnp
from jax import lax
from jax.experimental import pallas as pl
from jax.experimental.pallas import tpu as pltpu
from jax.experimental.pallas import tpu_sc as plsc

D_MODEL = 1024
LANES = 16
NUM_CORES = 2
NUM_SUBCORES = 16
NUM_WORKERS = NUM_CORES * NUM_SUBCORES


@functools.lru_cache(maxsize=None)
def _pos_encoding(seq_len: int) -> np.ndarray:
    pos = np.arange(seq_len, dtype=np.float32)[:, None]
    den = np.exp(
        (-np.arange(0, D_MODEL, 2, dtype=np.float32)) *
        np.float32(np.log(10000.0) / D_MODEL))
    pe = np.zeros((seq_len, D_MODEL), dtype=np.float32)
    pe[:, 0::2] = np.sin(pos * den)
    pe[:, 1::2] = np.cos(pos * den)
    return pe


@functools.lru_cache(maxsize=None)
def _build_sc_call(seq: int, batch: int):
    num_rows = seq * batch
    rows_per_w = num_rows // NUM_WORKERS
    cs = 4
    cr = cs * batch
    nchunk = rows_per_w // cr
    s_per_w = rows_per_w // batch
    ndb = D_MODEL // LANES
    scale = float(np.sqrt(D_MODEL))
    nbuf = 4
    lookahead = 2
    groups = nchunk // nbuf
    rem = nchunk % nbuf

    mesh = plsc.VectorSubcoreMesh(
        core_axis_name="c", subcore_axis_name="s")

    scratch = (
        [pltpu.VMEM((rows_per_w,), jnp.int32)]
        + [pltpu.VMEM((cr, D_MODEL), jnp.float32) for _ in range(nbuf)]
        + [pltpu.VMEM((cs, D_MODEL), jnp.float32) for _ in range(nbuf)]
        + [pltpu.SemaphoreType.DMA for _ in range(3 * nbuf)]
    )

    @functools.partial(
        pl.kernel,
        mesh=mesh,
        out_type=jax.ShapeDtypeStruct((seq, batch, D_MODEL), jnp.float32),
        scratch_types=scratch,
    )
    def embed(x_hbm, pe_hbm, table_hbm, out_hbm, idx_v, *bufs):
        rows_b = bufs[:nbuf]
        pe_b = bufs[nbuf:2 * nbuf]
        gsem = bufs[2 * nbuf:3 * nbuf]
        psem = bufs[3 * nbuf:4 * nbuf]
        ssem = bufs[4 * nbuf:5 * nbuf]

        wid = lax.axis_index("s") * NUM_CORES + lax.axis_index("c")
        base = wid * rows_per_w
        pltpu.sync_copy(x_hbm.at[pl.ds(base, rows_per_w)], idx_v)

        def gather_descs(c, k):
            return (
                pltpu.make_async_copy(
                    table_hbm.at[idx_v.at[pl.ds(c * cr, cr)]],
                    rows_b[k], gsem[k]),
                pltpu.make_async_copy(
                    pe_hbm.at[pl.ds(wid * s_per_w + c * cs, cs)],
                    pe_b[k], psem[k]),
            )

        def store_descs(c, k):
            s0 = wid * s_per_w + c * cs
            return [
                pltpu.make_async_copy(
                    rows_b[k].at[pl.ds(si * batch, batch)],
                    out_hbm.at[s0 + si], ssem[k])
                for si in range(cs)
            ]

        def start_fetch(c, k):
            g, p = gather_descs(c, k)
            g.start()
            p.start()

        for k in range(lookahead):
            start_fetch(k, k)

        def step(c, k):
            cp = c + lookahead
            kp = (k + lookahead) % nbuf

            @pl.when(cp < nchunk)
            def _prefetch():
                @pl.when(cp >= nbuf)
                def _drain_store():
                    for d in store_descs(cp - nbuf, kp):
                        d.wait()
                start_fetch(cp, kp)

            g, p = gather_descs(c, k)
            g.wait()
            p.wait()

            rows_v, pe_v = rows_b[k], pe_b[k]

            for si in range(cs):
                @plsc.parallel_loop(0, ndb // 2, unroll=4)
                def d_body(db2, _si=si):
                    for u in range(2):
                        dd = (db2 * 2 + u) * LANES
                        pe_vec = pe_v[_si, pl.ds(dd, LANES)]
                        for b in range(batch):
                            r = _si * batch + b
                            rows_v[r, pl.ds(dd, LANES)] = (
                                rows_v[r, pl.ds(dd, LANES)] * scale + pe_vec)

            for d in store_descs(c, k):
                d.start()

        def group_body(grp, carry):
            c0 = grp * nbuf
            for k in range(nbuf):
                step(c0 + k, k)
            return carry
        lax.fori_loop(0, groups, group_body, 0)

        for j in range(rem):
            step(groups * nbuf + j, j)

        for c in range(nchunk - min(nbuf, nchunk), nchunk):
            for d in store_descs(c, c % nbuf):
                d.wait()

    return embed


def kernel(x, table):
    seq, batch = x.shape
    xf = x.reshape(seq * batch).astype(jnp.int32)
    pe = jnp.asarray(_pos_encoding(seq))
    return _build_sc_call(seq, batch)(xf, pe, table)
